# Initial kernel scaffold; baseline (speedup 1.0000x reference)
import jax
import jax.numpy as jnp
from jax.experimental import pallas as pl


def kernel(coin_features, pvm, index, w):
    raise NotImplementedError("write your pallas kernel here")


# speedup vs baseline: 1.1836x; 1.1836x over previous
import jax
import jax.numpy as jnp
from jax import lax
from jax.experimental import pallas as pl
from jax.experimental.pallas import tpu as pltpu

F = 3
N = 64
P = 131072
W = 50
B = 1024

BB = 8
GRID = B // BB
PCHUNK = P // GRID


def _body(idx_ref, order_ref, lrow_ref, starts_ref,
          cf_ref, pvm_any_ref, pvm_ref, w_ref,
          x_ref, y_ref, lastw_ref, newpvm_ref,
          win, lw, sems, lw_sems, merge_sem):
    i = pl.program_id(0)
    nsteps = pl.num_programs(0)

    def start(step, slot):
        for j in range(BB):
            s = idx_ref[step * BB + j]
            a = jnp.minimum((s // 128) * 128, P - 2 * 128)
            pltpu.make_async_copy(
                cf_ref.at[:, :, pl.ds(a, 2 * 128)],
                win.at[slot, j],
                sems.at[slot],
            ).start()
            pltpu.make_async_copy(
                pvm_any_ref.at[pl.ds(s - 1, 1)],
                lw.at[slot, pl.ds(j, 1)],
                lw_sems.at[slot],
            ).start()

    @pl.when(i == 0)
    def _():
        start(0, 0)

    @pl.when(i + 1 < nsteps)
    def _():
        start(i + 1, (i + 1) % 2)

    newpvm_ref[...] = pvm_ref[...]

    def merge(k, carry):
        b = order_ref[k]
        row = lrow_ref[k]
        cp = pltpu.make_async_copy(
            w_ref.at[pl.ds(b, 1)],
            newpvm_ref.at[pl.ds(row, 1)],
            merge_sem,
        )
        cp.start()
        cp.wait()
        return carry

    lax.fori_loop(starts_ref[i], starts_ref[i + 1], merge, 0)

    slot = i % 2
    for j in range(BB):
        pltpu.make_async_copy(
            cf_ref.at[:, :, pl.ds(0, 2 * 128)],
            win.at[slot, j],
            sems.at[slot],
        ).wait()
    pltpu.make_async_copy(
        pvm_any_ref.at[pl.ds(0, BB)],
        lw.at[slot],
        lw_sems.at[slot],
    ).wait()

    for j in range(BB):
        s = idx_ref[i * BB + j]
        a = jnp.minimum((s // 128) * 128, P - 2 * 128)
        off = s - a
        wv = pltpu.roll(win[slot, j], (2 * 128 - off) % (2 * 128), axis=2)
        norm = wv[0:1, :, W - 1:W]
        x_ref[j] = wv[:, :, :W] / norm
        y_ref[j] = wv[:, :, W] / wv[0:1, :, W - 1]
    lastw_ref[...] = lw[slot]


def kernel(coin_features, pvm, index, w):
    index = index.astype(jnp.int32)
    order = jnp.argsort(index, stable=True).astype(jnp.int32)
    sorted_idx = index[order]
    lrow = (sorted_idx % PCHUNK).astype(jnp.int32)
    starts = jnp.searchsorted(
        sorted_idx, jnp.arange(GRID + 1, dtype=jnp.int32) * PCHUNK
    ).astype(jnp.int32)

    grid_spec = pltpu.PrefetchScalarGridSpec(
        num_scalar_prefetch=4,
        grid=(GRID,),
        in_specs=[
            pl.BlockSpec(memory_space=pl.ANY),
            pl.BlockSpec(memory_space=pl.ANY),
            pl.BlockSpec((PCHUNK, N), lambda i, *_: (i, 0)),
            pl.BlockSpec((B, N), lambda i, *_: (0, 0)),
        ],
        out_specs=[
            pl.BlockSpec((BB, F, N, W), lambda i, *_: (i, 0, 0, 0)),
            pl.BlockSpec((BB, F, N), lambda i, *_: (i, 0, 0)),
            pl.BlockSpec((BB, N), lambda i, *_: (i, 0)),
            pl.BlockSpec((PCHUNK, N), lambda i, *_: (i, 0)),
        ],
        scratch_shapes=[
            pltpu.VMEM((2, BB, F, N, 2 * 128), jnp.float32),
            pltpu.VMEM((2, BB, N), jnp.float32),
            pltpu.SemaphoreType.DMA((2,)),
            pltpu.SemaphoreType.DMA((2,)),
            pltpu.SemaphoreType.DMA,
        ],
    )
    X, y, last_w, new_pvm = pl.pallas_call(
        _body,
        grid_spec=grid_spec,
        out_shape=[
            jax.ShapeDtypeStruct((B, F, N, W), jnp.float32),
            jax.ShapeDtypeStruct((B, F, N), jnp.float32),
            jax.ShapeDtypeStruct((B, N), jnp.float32),
            jax.ShapeDtypeStruct((P, N), jnp.float32),
        ],
    )(index, order, lrow, starts, coin_features, pvm, pvm, w)
    return X, y, last_w, new_pvm
